# baseline (speedup 1.0000x reference)
import functools

import jax
import jax.numpy as jnp
from jax import lax
from jax.experimental import pallas as pl
from jax.experimental.pallas import tpu as pltpu
from jax.experimental.pallas import tpu_sc as plsc

_EMB_DIM = 1000000
_OUT_DIM = 64

_NC, _NS = 2, 16
_GATHER_WINDOW = 128


def _sc_gather(table, idx_flat):
    num_indices = idx_flat.shape[1]
    d = table.shape[1]
    mesh = plsc.VectorSubcoreMesh(core_axis_name="core",
                                  subcore_axis_name="subcore")

    @functools.partial(
        pl.kernel,
        out_type=jax.ShapeDtypeStruct((num_indices, d), table.dtype),
        mesh=mesh,
        compiler_params=pltpu.CompilerParams(use_tc_tiling_on_sc=False),
    )
    def gather_kernel(table_hbm, i_hbm, o_hbm):
        def body(i_vmem, o_vmem):
            pltpu.sync_copy(table_hbm.at[i_vmem.at[0]], o_vmem)

        pltpu.emit_pipeline(
            body,
            grid=(num_indices // _GATHER_WINDOW,),
            in_specs=[pl.BlockSpec((1, _GATHER_WINDOW),
                                   index_map=lambda i: (0, i))],
            out_specs=[pl.BlockSpec((_GATHER_WINDOW, d),
                                    index_map=lambda i: (i, 0))],
            core_axis_name=("core", "subcore"),
            dimension_semantics=(pltpu.PARALLEL,),
        )(i_hbm, o_hbm)

    return gather_kernel(table, idx_flat)


def _ln_body(e_ref, g_ref, b_ref, o_ref):
    e = e_ref[...]
    mean = jnp.mean(e, axis=-1, keepdims=True)
    c = e - mean
    var = jnp.mean(c * c, axis=-1, keepdims=True)
    inv = lax.rsqrt(var + 1e-5)
    o_ref[...] = c * inv * g_ref[...] + b_ref[...]


def _layernorm(emb, gamma, beta):
    n, d = emb.shape
    blk = 8192
    return pl.pallas_call(
        _ln_body,
        grid=(n // blk,),
        in_specs=[
            pl.BlockSpec((blk, d), lambda i: (i, 0)),
            pl.BlockSpec((1, d), lambda i: (0, 0)),
            pl.BlockSpec((1, d), lambda i: (0, 0)),
        ],
        out_specs=pl.BlockSpec((blk, d), lambda i: (i, 0)),
        out_shape=jax.ShapeDtypeStruct((n, d), emb.dtype),
    )(emb, gamma.reshape(1, d), beta.reshape(1, d))


def kernel(x, table, gamma, beta):
    batch, hist = x.shape
    row_sums = jnp.sum(x, axis=1, keepdims=True)
    x_norm = x / row_sums * (_EMB_DIM - 1)
    idx = jnp.clip(x_norm, 0, _EMB_DIM - 1).astype(jnp.int32)

    emb = _sc_gather(table, idx.reshape(1, batch * hist))
    out = _layernorm(emb, gamma, beta)
    return out.reshape(batch, hist, _OUT_DIM)
